# baseline (speedup 1.0000x reference)
import functools

import jax
import jax.numpy as jnp
from jax import lax
from jax.experimental import pallas as pl
from jax.experimental.pallas import tpu as pltpu
from jax.experimental.pallas import tpu_sc as plsc

VOCAB = 100000
HIDDEN = 1024
BATCH = 4
SEQ = 4096
EPS = 1e-5

NC = 2
NS = 16
LANES = 16
NW = NC * NS
NTOK = BATCH * SEQ
TOK_PER_W = NTOK // NW
C = 16
NCHUNK = TOK_PER_W // C
NVREG = HIDDEN // LANES
UNROLL = 8


def _rsqrt(x):
    i = lax.bitcast_convert_type(x, jnp.int32)
    y = lax.bitcast_convert_type(jnp.int32(0x5F3759DF) - (i >> 1), jnp.float32)
    for _ in range(3):
        y = y * (1.5 - 0.5 * x * y * y)
    return y


def _sc_kernel(ids_hbm, table_hbm, pos_hbm, gamma_hbm, beta_hbm, out_hbm,
               idx_all, gamma_v, beta_v,
               rows0, rows1, pos0, pos1, ob0, ob1,
               gsem0, gsem1, psem0, psem1, osem0, osem1):
    rows_b = (rows0, rows1)
    pos_b = (pos0, pos1)
    out_b = (ob0, ob1)
    gsem = (gsem0, gsem1)
    psem = (psem0, psem1)
    osem = (osem0, osem1)

    wid = lax.axis_index("s") * NC + lax.axis_index("c")
    base = wid * TOK_PER_W
    s0 = base % SEQ

    pltpu.sync_copy(ids_hbm.at[pl.ds(base, TOK_PER_W)], idx_all)
    pltpu.sync_copy(gamma_hbm, gamma_v)
    pltpu.sync_copy(beta_hbm, beta_v)

    def start_fetch(g, slot):
        idx_vec = idx_all[pl.ds(g * C, C)]
        pltpu.async_copy(table_hbm.at[idx_vec], rows_b[slot], gsem[slot])
        pltpu.async_copy(pos_hbm.at[pl.ds(s0 + g * C, C)], pos_b[slot],
                         psem[slot])

    def wait_fetch(g, slot):
        idx_vec = idx_all[pl.ds(g * C, C)]
        pltpu.make_async_copy(table_hbm.at[idx_vec], rows_b[slot],
                              gsem[slot]).wait()
        pltpu.make_async_copy(pos_hbm.at[pl.ds(s0 + g * C, C)], pos_b[slot],
                              psem[slot]).wait()

    def start_out(g, slot):
        pltpu.async_copy(out_b[slot], out_hbm.at[pl.ds(base + g * C, C)],
                         osem[slot])

    def wait_out(g, slot):
        pltpu.make_async_copy(out_b[slot], out_hbm.at[pl.ds(base + g * C, C)],
                              osem[slot]).wait()

    def compute_chunk(slot):
        rows = rows_b[slot]
        posb = pos_b[slot]
        ob = out_b[slot]

        def tok0(t, _):
            def p0(jj, carry):
                for u in range(UNROLL):
                    sl = pl.ds(jj * (UNROLL * LANES) + u * LANES, LANES)
                    ob[t, sl] = rows[t, sl] + posb[t, sl]
                return carry

            lax.fori_loop(0, NVREG // UNROLL, p0, 0)
            return 0

        lax.fori_loop(0, C, tok0, 0)

        lane = jnp.arange(LANES, dtype=jnp.int32)
        zero = jnp.zeros((LANES,), jnp.float32)

        def colgrp(cc, carry):
            acc, acc2 = carry
            for u in range(UNROLL):
                c = cc * UNROLL + u
                col = jnp.full((LANES,), c, jnp.int32)
                x = plsc.load_gather(ob, [lane, col])
                acc = acc + x
                acc2 = acc2 + x * x
            return acc, acc2

        acc, acc2 = lax.fori_loop(0, HIDDEN // UNROLL, colgrp, (zero, zero))
        mean = acc * (1.0 / HIDDEN)
        var = acc2 * (1.0 / HIDDEN) - mean * mean
        inv = _rsqrt(var + EPS)

        def tok2(t, _):
            tt = jnp.full((LANES,), t, jnp.int32)
            m = mean.at[tt].get(mode="promise_in_bounds")
            s = inv.at[tt].get(mode="promise_in_bounds")

            def p2(jj, carry):
                for u in range(UNROLL):
                    sl = pl.ds(jj * (UNROLL * LANES) + u * LANES, LANES)
                    x = ob[t, sl]
                    ob[t, sl] = (x - m) * s * gamma_v[sl] + beta_v[sl]
                return carry

            lax.fori_loop(0, NVREG // UNROLL, p2, 0)
            return 0

        lax.fori_loop(0, C, tok2, 0)

    start_fetch(0, 0)
    start_fetch(1, 1)

    def pair_body(p, carry):
        for slot in range(2):
            g = 2 * p + slot

            wait_fetch(g, slot)

            @pl.when(g >= 2)
            def _():
                wait_out(g - 2, slot)

            compute_chunk(slot)
            start_out(g, slot)

            @pl.when(g + 2 < NCHUNK)
            def _():
                start_fetch(g + 2, slot)
        return carry

    lax.fori_loop(0, NCHUNK // 2, pair_body, 0)

    wait_out(NCHUNK - 2, 0)
    wait_out(NCHUNK - 1, 1)


@jax.jit
def _run(ids_flat, word_table, pos_table, gamma, beta):
    mesh = plsc.VectorSubcoreMesh(core_axis_name="c", subcore_axis_name="s")
    fn = pl.kernel(
        _sc_kernel,
        mesh=mesh,
        compiler_params=pltpu.CompilerParams(
            use_tc_tiling_on_sc=False, needs_layout_passes=False),
        out_type=jax.ShapeDtypeStruct((NTOK, HIDDEN), jnp.float32),
        scratch_types=[
            pltpu.VMEM((TOK_PER_W,), jnp.int32),
            pltpu.VMEM((HIDDEN,), jnp.float32),
            pltpu.VMEM((HIDDEN,), jnp.float32),
            pltpu.VMEM((C, HIDDEN), jnp.float32),
            pltpu.VMEM((C, HIDDEN), jnp.float32),
            pltpu.VMEM((C, HIDDEN), jnp.float32),
            pltpu.VMEM((C, HIDDEN), jnp.float32),
            pltpu.VMEM((C, HIDDEN), jnp.float32),
            pltpu.VMEM((C, HIDDEN), jnp.float32),
            pltpu.SemaphoreType.DMA,
            pltpu.SemaphoreType.DMA,
            pltpu.SemaphoreType.DMA,
            pltpu.SemaphoreType.DMA,
            pltpu.SemaphoreType.DMA,
            pltpu.SemaphoreType.DMA,
        ],
    )
    return fn(ids_flat, word_table, pos_table, gamma, beta)


def kernel(input_ids, word_table, pos_table, gamma, beta):
    ids_flat = input_ids.reshape(-1).astype(jnp.int32)
    out = _run(ids_flat, word_table, pos_table, gamma, beta)
    return out.reshape(BATCH, SEQ, HIDDEN)


# speedup vs baseline: 9.9935x; 9.9935x over previous
import jax
import jax.numpy as jnp
from jax import lax
from jax.experimental import pallas as pl
from jax.experimental.pallas import tpu as pltpu
from jax.experimental.pallas import tpu_sc as plsc

VOCAB = 100000
HIDDEN = 1024
BATCH = 4
SEQ = 4096
EPS = 1e-5

NC = 2
NS = 16
LANES = 16
NW = NC * NS
NTOK = BATCH * SEQ
TOK_PER_W = NTOK // NW
C = 16
NCHUNK = TOK_PER_W // C
KB = HIDDEN // 128


def _rsqrt(x):
    i = lax.bitcast_convert_type(x, jnp.int32)
    y = lax.bitcast_convert_type(jnp.int32(0x5F3759DF) - (i >> 1), jnp.float32)
    for _ in range(3):
        y = y * (1.5 - 0.5 * x * y * y)
    return y


def _sc_kernel(ids_hbm, table_hbm, pos_hbm, gamma_hbm, beta_hbm, out_hbm,
               idx_all, gamma_v, beta_v, sb, sb2,
               rows0, rows1, pos0, pos1, ob0, ob1, ixb0, ixb1,
               gsem0, gsem1, psem0, psem1, osem0, osem1):
    rows_b = (rows0, rows1)
    pos_b = (pos0, pos1)
    out_b = (ob0, ob1)
    ixb_b = (ixb0, ixb1)
    gsem = (gsem0, gsem1)
    psem = (psem0, psem1)
    osem = (osem0, osem1)

    wid = lax.axis_index("s") * NC + lax.axis_index("c")
    base = wid * TOK_PER_W
    s0 = base % SEQ

    pltpu.sync_copy(ids_hbm.at[pl.ds(base, TOK_PER_W)], idx_all)
    pltpu.sync_copy(gamma_hbm, gamma_v)
    pltpu.sync_copy(beta_hbm, beta_v)

    def start_fetch(g, slot):
        ids_vec = idx_all[pl.ds(g * C, C)]
        basev = (lax.shift_right_logical(ids_vec, 3) << 6) \
            + jnp.bitwise_and(ids_vec, 7)
        ixb = ixb_b[slot]
        for k in range(KB):
            ixb[pl.ds(k * C, C)] = basev + (k * 8)
        pltpu.async_copy(table_hbm.at[ixb], rows_b[slot], gsem[slot])
        ptr0 = s0 // 8 + g * 2
        pltpu.async_copy(pos_hbm.at[pl.ds(ptr0, 2)], pos_b[slot], psem[slot])

    def wait_fetch(g, slot):
        pltpu.make_async_copy(table_hbm.at[ixb_b[slot]], rows_b[slot],
                              gsem[slot]).wait()
        ptr0 = s0 // 8 + g * 2
        pltpu.make_async_copy(pos_hbm.at[pl.ds(ptr0, 2)], pos_b[slot],
                              psem[slot]).wait()

    def start_out(g, slot):
        otr0 = base // 8 + g * 2
        pltpu.async_copy(out_b[slot], out_hbm.at[pl.ds(otr0, 2)], osem[slot])

    def wait_out(g, slot):
        otr0 = base // 8 + g * 2
        pltpu.make_async_copy(out_b[slot], out_hbm.at[pl.ds(otr0, 2)],
                              osem[slot]).wait()

    zero = jnp.zeros((LANES,), jnp.float32)
    lane = jnp.arange(LANES, dtype=jnp.int32)

    def compute_chunk(slot):
        rows = rows_b[slot]
        posb = pos_b[slot]
        ob = out_b[slot]

        @plsc.parallel_loop(0, C)
        def tok0(t):
            a = t >> 3
            r = jnp.bitwise_and(t, 7)

            @plsc.parallel_loop(0, KB * 8, unroll=8,
                                carry=(zero, zero, zero, zero))
            def ebody(e, carry):
                aa, ab, a2a, a2b = carry
                k = e >> 3
                u = jnp.bitwise_and(e, 7)
                sl = pl.ds(u * 16, 16)
                x = rows[k * C + t, sl] + posb[a, k, r, sl]
                ob[a, k, r, sl] = x
                return ab, aa + x, a2b, a2a + x * x

            aa, ab, a2a, a2b = ebody
            sb[t, pl.ds(0, 16)] = aa + ab
            sb2[t, pl.ds(0, 16)] = a2a + a2b

        tot = zero
        tot2 = zero
        for c in range(16):
            cc = jnp.full((LANES,), c, jnp.int32)
            tot = tot + plsc.load_gather(sb, [lane, cc])
            tot2 = tot2 + plsc.load_gather(sb2, [lane, cc])
        mean = tot * (1.0 / HIDDEN)
        var = tot2 * (1.0 / HIDDEN) - mean * mean
        inv = _rsqrt(var + EPS)

        @plsc.parallel_loop(0, KB, unroll=2)
        def kbody2(k):
            for th in range(2):
                msp = []
                ssp = []
                for tt in range(8):
                    ct = jnp.full((LANES,), th * 8 + tt, jnp.int32)
                    msp.append(mean.at[ct].get(mode="promise_in_bounds"))
                    ssp.append(inv.at[ct].get(mode="promise_in_bounds"))
                for u in range(8):
                    sl = pl.ds(k * 128 + u * 16, 16)
                    gv = gamma_v[sl]
                    bv = beta_v[sl]
                    usl = pl.ds(u * 16, 16)
                    for tt in range(8):
                        x = ob[th, k, tt, usl]
                        ob[th, k, tt, usl] = (x - msp[tt]) * ssp[tt] * gv + bv

    start_fetch(0, 0)
    start_fetch(1, 1)

    def pair_body(p, carry):
        for slot in range(2):
            g = 2 * p + slot

            wait_fetch(g, slot)

            @pl.when(g >= 2)
            def _():
                wait_out(g - 2, slot)

            compute_chunk(slot)
            start_out(g, slot)

            @pl.when(g + 2 < NCHUNK)
            def _():
                start_fetch(g + 2, slot)
        return carry

    lax.fori_loop(0, NCHUNK // 2, pair_body, 0)

    wait_out(NCHUNK - 2, 0)
    wait_out(NCHUNK - 1, 1)


def _run(ids_flat, table4, pos4, gamma, beta):
    mesh = plsc.VectorSubcoreMesh(core_axis_name="c", subcore_axis_name="s")
    fn = pl.kernel(
        _sc_kernel,
        mesh=mesh,
        compiler_params=pltpu.CompilerParams(
            use_tc_tiling_on_sc=False, needs_layout_passes=False),
        out_type=jax.ShapeDtypeStruct((NTOK // 8, KB, 8, 128), jnp.float32),
        scratch_types=[
            pltpu.VMEM((TOK_PER_W,), jnp.int32),
            pltpu.VMEM((HIDDEN,), jnp.float32),
            pltpu.VMEM((HIDDEN,), jnp.float32),
            pltpu.VMEM((C, 17), jnp.float32),
            pltpu.VMEM((C, 17), jnp.float32),
            pltpu.VMEM((C * KB, 128), jnp.float32),
            pltpu.VMEM((C * KB, 128), jnp.float32),
            pltpu.VMEM((2, KB, 8, 128), jnp.float32),
            pltpu.VMEM((2, KB, 8, 128), jnp.float32),
            pltpu.VMEM((2, KB, 8, 128), jnp.float32),
            pltpu.VMEM((2, KB, 8, 128), jnp.float32),
            pltpu.VMEM((C * KB,), jnp.int32),
            pltpu.VMEM((C * KB,), jnp.int32),
            pltpu.SemaphoreType.DMA,
            pltpu.SemaphoreType.DMA,
            pltpu.SemaphoreType.DMA,
            pltpu.SemaphoreType.DMA,
            pltpu.SemaphoreType.DMA,
            pltpu.SemaphoreType.DMA,
        ],
    )
    return fn(ids_flat, table4, pos4, gamma, beta)


def kernel(input_ids, word_table, pos_table, gamma, beta):
    ids_flat = input_ids.reshape(-1).astype(jnp.int32)
    table4 = (word_table.reshape(VOCAB // 8, 8, KB, 128)
              .transpose(0, 2, 1, 3).reshape(VOCAB * KB, 128))
    pos4 = (pos_table.reshape(SEQ // 8, 8, KB, 128)
            .transpose(0, 2, 1, 3))
    out4 = _run(ids_flat, table4, pos4, gamma, beta)
    return (out4.transpose(0, 2, 1, 3)
            .reshape(BATCH, SEQ, HIDDEN))
